# Initial kernel scaffold; baseline (speedup 1.0000x reference)
import jax
import jax.numpy as jnp
from jax.experimental import pallas as pl


def kernel(x, edge_index, W1, b1, W2, b2):
    raise NotImplementedError("write your pallas kernel here")


# speedup vs baseline: 156.5517x; 156.5517x over previous
import functools

import jax
import jax.numpy as jnp
from jax import lax
from jax.experimental import pallas as pl
from jax.experimental.pallas import tpu as pltpu
from jax.experimental.pallas import tpu_sc as plsc

NC, NS = 2, 16
NW = NC * NS
LB = 128
KCH = 32


def _edge_pass(rows_pt, nch, npad, n_sl, with_gather):
  mesh = plsc.VectorSubcoreMesh(core_axis_name="c", subcore_axis_name="s")
  scratch = [
      pltpu.VMEM((KCH, LB), jnp.int32),
      pltpu.VMEM((KCH * LB,), jnp.float32),
      pltpu.VMEM((n_sl,), jnp.float32),
      pltpu.SemaphoreType.DMA,
      pltpu.SemaphoreType.DMA,
      pltpu.VMEM_SHARED((npad,), jnp.float32),
  ]
  if with_gather:
    scratch += [
        pltpu.VMEM((KCH, LB), jnp.int32),
        pltpu.VMEM_SHARED((npad,), jnp.float32),
    ]

  def body(*refs):
    if with_gather:
      (src2, dst2, tbl_h, out_h,
       didx, vals, bnc, gsem, ssem, acc_sh, sidx, tbl_sh) = refs
    else:
      (dst2, out_h, didx, vals, bnc, gsem, ssem, acc_sh) = refs
    c = lax.axis_index("c")
    s = lax.axis_index("s")
    wid = c * NS + s

    def _z(i, carry):
      bnc[pl.ds(i * 16, 16)] = jnp.zeros((16,), jnp.float32)
      return carry
    lax.fori_loop(0, n_sl // 16, _z, None)
    pltpu.sync_copy(bnc, acc_sh.at[pl.ds(s * n_sl, n_sl)])
    if with_gather:
      pltpu.sync_copy(tbl_h.at[pl.ds(s * n_sl, n_sl)], bnc)
      pltpu.sync_copy(bnc, tbl_sh.at[pl.ds(s * n_sl, n_sl)])
    else:
      def _o(i, carry):
        vals[pl.ds(i * 16, 16)] = jnp.full((16,), 1.0, jnp.float32)
        return carry
      lax.fori_loop(0, (KCH * LB) // 16, _o, None)
    plsc.subcore_barrier()

    row0 = wid * rows_pt

    def chunk(ch, carry):
      base = row0 + ch * KCH
      pltpu.sync_copy(dst2.at[pl.ds(base, KCH)], didx)
      if with_gather:
        pltpu.sync_copy(src2.at[pl.ds(base, KCH)], sidx)
        gds = [
            pltpu.async_copy(tbl_sh.at[sidx.at[j]],
                             vals.at[pl.ds(j * LB, LB)], gsem)
            for j in range(KCH)
        ]
        for d in gds:
          d.wait()
      sds = [
          pltpu.async_copy(vals.at[pl.ds(j * LB, LB)],
                           acc_sh.at[didx.at[j]], ssem, add=True)
          for j in range(KCH)
      ]
      for d in sds:
        d.wait()
      return carry

    lax.fori_loop(0, nch, chunk, None)
    plsc.subcore_barrier()

    pltpu.sync_copy(acc_sh.at[pl.ds(s * n_sl, n_sl)], bnc)
    pltpu.sync_copy(bnc, out_h.at[pl.ds(c * npad + s * n_sl, n_sl)])

  return pl.kernel(
      body,
      out_type=jax.ShapeDtypeStruct((NC * npad,), jnp.float32),
      mesh=mesh,
      scratch_types=scratch,
  )


def _tc_prep(rows):
  def body(degp, xp, dis, p):
    deg = degp[0] + degp[1] + 1.0
    d = lax.rsqrt(deg)
    d = d * (1.5 - 0.5 * deg * d * d)
    d = d * (1.5 - 0.5 * deg * d * d)
    dis[...] = d
    p[...] = d * xp[...]

  return pl.pallas_call(
      body,
      out_shape=[jax.ShapeDtypeStruct((rows, 128), jnp.float32)] * 2,
  )


def _tc_act(rows, width):
  def body(sp, dis, xp, w1, b1, w2, q):
    d = dis[...]
    s1 = d * (sp[0] + sp[1] + d * xp[...])
    acc = jnp.zeros_like(s1)
    for k in range(width):
      r = jnp.maximum(s1 * w1[k] + b1[k], 0.0)
      r = r.astype(jnp.bfloat16).astype(jnp.float32)
      w2k = w2[k].astype(jnp.bfloat16).astype(jnp.float32)
      acc = acc + r * w2k
    q[...] = d * acc

  smem = pl.BlockSpec(memory_space=pltpu.SMEM)
  return pl.pallas_call(
      body,
      in_specs=[pl.BlockSpec((NC, rows, 128), lambda: (0, 0, 0)),
                pl.BlockSpec((rows, 128), lambda: (0, 0)),
                pl.BlockSpec((rows, 128), lambda: (0, 0)),
                smem, smem, smem],
      out_shape=jax.ShapeDtypeStruct((rows, 128), jnp.float32),
  )


def _tc_fin(rows, n):
  def body(sp, dis, q, b2, out):
    d = dis[...]
    v = d * (sp[0] + sp[1]) + d * q[...] + b2[0]
    rid = lax.broadcasted_iota(jnp.int32, (rows, 128), 0)
    cid = lax.broadcasted_iota(jnp.int32, (rows, 128), 1)
    v = jnp.where(rid * 128 + cid < n, v, -jnp.inf)
    out[...] = jnp.max(v).reshape(1, 1)

  smem = pl.BlockSpec(memory_space=pltpu.SMEM)
  return pl.pallas_call(
      body,
      in_specs=[pl.BlockSpec((NC, rows, 128), lambda: (0, 0, 0)),
                pl.BlockSpec((rows, 128), lambda: (0, 0)),
                pl.BlockSpec((rows, 128), lambda: (0, 0)),
                smem],
      out_shape=jax.ShapeDtypeStruct((1, 1), jnp.float32),
  )


def kernel(x, edge_index, W1, b1, W2, b2):
  n = x.shape[0]
  e = edge_index.shape[1]
  width = W1.shape[1]

  n_sl = -(-(n + 1) // (NS * 8)) * 8
  npad = NS * n_sl
  rows = npad // 128
  rows_pt = -(-e // (NW * LB))
  rows_pt = -(-rows_pt // KCH) * KCH
  nch = rows_pt // KCH
  ep = NW * rows_pt * LB

  src = jnp.pad(edge_index[0], (0, ep - e), constant_values=n)
  dst = jnp.pad(edge_index[1], (0, ep - e), constant_values=n)
  src2 = src.reshape(ep // LB, LB)
  dst2 = dst.reshape(ep // LB, LB)
  xp = jnp.pad(x[:, 0], (0, npad - n)).reshape(rows, 128)

  conv_pass = _edge_pass(rows_pt, nch, npad, n_sl, True)

  degp = _edge_pass(rows_pt, nch, npad, n_sl, False)(dst2)
  dis, p = _tc_prep(rows)(degp.reshape(NC, rows, 128), xp)
  sp1 = conv_pass(src2, dst2, p.reshape(npad))
  q = _tc_act(rows, width)(sp1.reshape(NC, rows, 128), dis, xp,
                           W1.reshape(width), b1, W2.reshape(width))
  sp2 = conv_pass(src2, dst2, q.reshape(npad))
  return _tc_fin(rows, n)(sp2.reshape(NC, rows, 128), dis, q, b2)


# speedup vs baseline: 212.8339x; 1.3595x over previous
import functools

import jax
import jax.numpy as jnp
from jax import lax
from jax.experimental import pallas as pl
from jax.experimental.pallas import tpu as pltpu
from jax.experimental.pallas import tpu_sc as plsc

NC, NS = 2, 16
NW = NC * NS
LB = 128
KCH = 16
LANES = 16


def _mesh():
  return plsc.VectorSubcoreMesh(core_axis_name="c", subcore_axis_name="s")


def _zero_acc_slice(bnc, acc_sh, s, n_sl):
  def _z(i, carry):
    bnc[pl.ds(i * LANES, LANES)] = jnp.zeros((LANES,), jnp.float32)
    return carry
  lax.fori_loop(0, n_sl // LANES, _z, None)
  pltpu.sync_copy(bnc, acc_sh.at[pl.ds(s * n_sl, n_sl)])


def _write_out(bnc, acc_sh, out_h, c, s, n_sl, npad):
  pltpu.sync_copy(acc_sh.at[pl.ds(s * n_sl, n_sl)], bnc)
  pltpu.sync_copy(bnc, out_h.at[pl.ds(c * npad + s * n_sl, n_sl)])


def _deg_pass(rows_pt, nch, npad, n_sl):
  scratch = [
      pltpu.VMEM((KCH, LB), jnp.int32),
      pltpu.VMEM((KCH * LB,), jnp.float32),
      pltpu.VMEM((n_sl,), jnp.float32),
      pltpu.SemaphoreType.DMA,
      pltpu.VMEM_SHARED((npad,), jnp.float32),
  ]

  def body(dst2, out_h, didx, vals, bnc, ssem, acc_sh):
    c = lax.axis_index("c")
    s = lax.axis_index("s")
    wid = c * NS + s
    _zero_acc_slice(bnc, acc_sh, s, n_sl)

    def _o(i, carry):
      vals[pl.ds(i * LANES, LANES)] = jnp.full((LANES,), 1.0, jnp.float32)
      return carry
    lax.fori_loop(0, (KCH * LB) // LANES, _o, None)
    plsc.subcore_barrier()

    row0 = wid * rows_pt

    def chunk(ch, carry):
      pltpu.sync_copy(dst2.at[pl.ds(row0 + ch * KCH, KCH)], didx)
      sds = [
          pltpu.async_copy(vals.at[pl.ds(j * LB, LB)],
                           acc_sh.at[didx.at[j]], ssem, add=True)
          for j in range(KCH)
      ]
      for d in sds:
        d.wait()
      return carry

    lax.fori_loop(0, nch, chunk, None)
    plsc.subcore_barrier()
    _write_out(bnc, acc_sh, out_h, c, s, n_sl, npad)

  return pl.kernel(
      body,
      out_type=jax.ShapeDtypeStruct((NC * npad,), jnp.float32),
      mesh=_mesh(),
      scratch_types=scratch,
  )


def _conv_pass(rows_pt, nch, npad, n_sl):
  scratch = [
      pltpu.VMEM((npad,), jnp.float32),
      pltpu.VMEM((KCH, LB), jnp.int32),
      pltpu.VMEM((n_sl,), jnp.float32),
      pltpu.VMEM_SHARED((npad,), jnp.float32),
  ]
  for _ in range(2):
    scratch += [
        pltpu.VMEM((KCH, LB), jnp.int32),
        pltpu.VMEM((KCH, LB), jnp.float32),
        pltpu.SemaphoreType.DMA,
    ]

  def body(src2, dst2, tbl_h, out_h, tblv, sidx, bnc, acc_sh,
           didx0, vals0, ssem0, didx1, vals1, ssem1):
    c = lax.axis_index("c")
    s = lax.axis_index("s")
    wid = c * NS + s
    _zero_acc_slice(bnc, acc_sh, s, n_sl)
    pltpu.sync_copy(tbl_h, tblv)
    plsc.subcore_barrier()

    row0 = wid * rows_pt
    sets = ((didx0, vals0, ssem0), (didx1, vals1, ssem1))

    def scatters(didx, vals, ssem, issue):
      if issue:
        return [
            pltpu.async_copy(vals.at[j], acc_sh.at[didx.at[j]], ssem, add=True)
            for j in range(KCH)
        ]
      return [
          pltpu.make_async_copy(vals.at[j], acc_sh.at[didx.at[j]], ssem)
          for j in range(KCH)
      ]

    def do_chunk(t, ch, didx, vals, ssem):
      @pl.when(t > 0)
      def _():
        for d in scatters(didx, vals, ssem, issue=False):
          d.wait()
      pltpu.sync_copy(src2.at[pl.ds(ch, KCH)], sidx)
      pltpu.sync_copy(dst2.at[pl.ds(ch, KCH)], didx)
      for j in range(KCH):
        for b in range(LB // LANES):
          iv = sidx[j, pl.ds(b * LANES, LANES)]
          vals[j, pl.ds(b * LANES, LANES)] = plsc.load_gather(tblv, [iv])
      scatters(didx, vals, ssem, issue=True)

    def pair(t, carry):
      base = row0 + t * (2 * KCH)
      do_chunk(t, base, *sets[0])
      do_chunk(t, base + KCH, *sets[1])
      return carry

    lax.fori_loop(0, nch // 2, pair, None)
    for didx, vals, ssem in sets:
      for d in scatters(didx, vals, ssem, issue=False):
        d.wait()
    plsc.subcore_barrier()
    _write_out(bnc, acc_sh, out_h, c, s, n_sl, npad)

  return pl.kernel(
      body,
      out_type=jax.ShapeDtypeStruct((NC * npad,), jnp.float32),
      mesh=_mesh(),
      scratch_types=scratch,
      compiler_params=pltpu.CompilerParams(needs_layout_passes=False),
  )


def _tc_prep(rows):
  def body(degp, xp, dis, p):
    deg = degp[0] + degp[1] + 1.0
    d = lax.rsqrt(deg)
    d = d * (1.5 - 0.5 * deg * d * d)
    d = d * (1.5 - 0.5 * deg * d * d)
    dis[...] = d
    p[...] = d * xp[...]

  return pl.pallas_call(
      body,
      out_shape=[jax.ShapeDtypeStruct((rows, 128), jnp.float32)] * 2,
  )


def _tc_act(rows, width):
  def body(sp, dis, xp, w1, b1, w2, q):
    d = dis[...]
    s1 = d * (sp[0] + sp[1] + d * xp[...])
    acc = jnp.zeros_like(s1)
    for k in range(width):
      r = jnp.maximum(s1 * w1[k] + b1[k], 0.0)
      r = r.astype(jnp.bfloat16).astype(jnp.float32)
      w2k = w2[k].astype(jnp.bfloat16).astype(jnp.float32)
      acc = acc + r * w2k
    q[...] = d * acc

  smem = pl.BlockSpec(memory_space=pltpu.SMEM)
  return pl.pallas_call(
      body,
      in_specs=[pl.BlockSpec((NC, rows, 128), lambda: (0, 0, 0)),
                pl.BlockSpec((rows, 128), lambda: (0, 0)),
                pl.BlockSpec((rows, 128), lambda: (0, 0)),
                smem, smem, smem],
      out_shape=jax.ShapeDtypeStruct((rows, 128), jnp.float32),
  )


def _tc_fin(rows, n):
  def body(sp, dis, q, b2, out):
    d = dis[...]
    v = d * (sp[0] + sp[1]) + d * q[...] + b2[0]
    rid = lax.broadcasted_iota(jnp.int32, (rows, 128), 0)
    cid = lax.broadcasted_iota(jnp.int32, (rows, 128), 1)
    v = jnp.where(rid * 128 + cid < n, v, -jnp.inf)
    out[...] = jnp.max(v).reshape(1, 1)

  smem = pl.BlockSpec(memory_space=pltpu.SMEM)
  return pl.pallas_call(
      body,
      in_specs=[pl.BlockSpec((NC, rows, 128), lambda: (0, 0, 0)),
                pl.BlockSpec((rows, 128), lambda: (0, 0)),
                pl.BlockSpec((rows, 128), lambda: (0, 0)),
                smem],
      out_shape=jax.ShapeDtypeStruct((1, 1), jnp.float32),
  )


def kernel(x, edge_index, W1, b1, W2, b2):
  n = x.shape[0]
  e = edge_index.shape[1]
  width = W1.shape[1]

  n_sl = -(-(n + 1) // (NS * 8)) * 8
  npad = NS * n_sl
  rows = npad // 128
  rows_pt = -(-e // (NW * LB))
  rows_pt = -(-rows_pt // (2 * KCH)) * (2 * KCH)
  nch = rows_pt // KCH
  ep = NW * rows_pt * LB

  src = jnp.pad(edge_index[0], (0, ep - e), constant_values=n)
  dst = jnp.pad(edge_index[1], (0, ep - e), constant_values=n)
  src2 = src.reshape(ep // LB, LB)
  dst2 = dst.reshape(ep // LB, LB)
  xp = jnp.pad(x[:, 0], (0, npad - n)).reshape(rows, 128)

  conv = _conv_pass(rows_pt, nch, npad, n_sl)

  degp = _deg_pass(rows_pt, nch, npad, n_sl)(dst2)
  dis, p = _tc_prep(rows)(degp.reshape(NC, rows, 128), xp)
  sp1 = conv(src2, dst2, p.reshape(npad))
  q = _tc_act(rows, width)(sp1.reshape(NC, rows, 128), dis, xp,
                           W1.reshape(width), b1, W2.reshape(width))
  sp2 = conv(src2, dst2, q.reshape(npad))
  return _tc_fin(rows, n)(sp2.reshape(NC, rows, 128), dis, q, b2)
